# Initial kernel scaffold; baseline (speedup 1.0000x reference)
import jax
import jax.numpy as jnp
from jax.experimental import pallas as pl


def kernel(f_in0, f_in1, pos, params, rigid_transforms, rigid_groups, edge_index, residue_type, transforms_dep, groups_dep):
    raise NotImplementedError("write your pallas kernel here")


# speedup vs baseline: 2.8037x; 2.8037x over previous
import functools

import jax
import jax.numpy as jnp
from jax import lax
from jax.experimental import pallas as pl
from jax.experimental.pallas import tpu as pltpu
from jax.experimental.pallas import tpu_sc as plsc

N = 10000
E = 320000
D = 128
NRES = 21
MAX_RIGID = 8
NATOM = 14

NC = 2
NS = 16
NW = NC * NS

NP = 10240
ROWS_PER_SUB = NP // NS

EDGE_CHUNK = 128
CHUNKS_PER_W = 80
E_PAD = NW * CHUNKS_PER_W * EDGE_CHUNK

GROWS_PER_W = NP // NW
GCHUNK = 80
GCHUNKS = GROWS_PER_W // GCHUNK
TBL = 96 + 42 + 8 + 14
TBL_P = 256

def _sc_mesh():
    return plsc.VectorSubcoreMesh(
        core_axis_name="c", subcore_axis_name="s", num_cores=NC, num_subcores=NS)


@functools.cache
def _build_seg_sum():
    @functools.partial(
        pl.kernel,
        out_type=jax.ShapeDtypeStruct((NC, NP, D), jnp.float32),
        mesh=_sc_mesh(),
        scratch_types=[
            pltpu.VMEM((CHUNKS_PER_W, EDGE_CHUNK), jnp.int32),
            pltpu.VMEM((CHUNKS_PER_W, EDGE_CHUNK), jnp.int32),
            pltpu.VMEM((EDGE_CHUNK, D), jnp.float32),
            pltpu.VMEM_SHARED((NP, D), jnp.float32),
            pltpu.SemaphoreType.DMA,
        ],
    )
    def seg_sum(x_hbm, src_hbm, dst_hbm, zeros_hbm, out_hbm,
                src_v, dst_v, rows_v, acc_sh, sem):
        cid = lax.axis_index("c")
        sid = lax.axis_index("s")
        wid = cid * NS + sid

        pltpu.sync_copy(zeros_hbm, acc_sh.at[pl.ds(sid * ROWS_PER_SUB, ROWS_PER_SUB)])
        pltpu.sync_copy(src_hbm.at[wid], src_v)
        pltpu.sync_copy(dst_hbm.at[wid], dst_v)
        plsc.subcore_barrier()

        @pl.loop(0, CHUNKS_PER_W)
        def _chunks(c):
            pltpu.async_copy(x_hbm.at[src_v.at[c]], rows_v, sem).wait()
            pltpu.sync_copy(rows_v, acc_sh.at[dst_v.at[c]], add=True)

        plsc.subcore_barrier()
        pltpu.sync_copy(
            acc_sh.at[pl.ds(sid * ROWS_PER_SUB, ROWS_PER_SUB)],
            out_hbm.at[cid, pl.ds(sid * ROWS_PER_SUB, ROWS_PER_SUB)],
        )

    return seg_sum


def _seg_sum(x, src_p, dst_p, zeros_hbm):
    return _build_seg_sum()(x, src_p, dst_p, zeros_hbm)


@functools.cache
def _build_table_gather():
    @functools.partial(
        pl.kernel,
        out_type=jax.ShapeDtypeStruct((NP, TBL_P), jnp.float32),
        mesh=_sc_mesh(),
        scratch_types=[
            pltpu.VMEM((GCHUNKS, GCHUNK), jnp.int32),
            pltpu.VMEM((GCHUNK, TBL_P), jnp.float32),
            pltpu.SemaphoreType.DMA,
        ],
    )
    def table_gather(tbl_hbm, idx_hbm, out_hbm, idx_v, rows_v, sem):
        cid = lax.axis_index("c")
        sid = lax.axis_index("s")
        wid = cid * NS + sid
        pltpu.sync_copy(idx_hbm.at[wid], idx_v)

        @pl.loop(0, GCHUNKS)
        def _chunks(c):
            pltpu.async_copy(tbl_hbm.at[idx_v.at[c]], rows_v, sem).wait()
            base = wid * GROWS_PER_W + c * GCHUNK
            pltpu.sync_copy(rows_v, out_hbm.at[pl.ds(base, GCHUNK)])

    return table_gather


def _table_gather(tbl, ridx):
    return _build_table_gather()(tbl, ridx)


CONV_BLK = 2048


def _conv_body(acc_ref, x_ref, wm_ref, ws_ref, b_ref, o_ref, *, act):
    agg = acc_ref[0] + acc_ref[1]
    out = jnp.dot(agg, wm_ref[...], preferred_element_type=jnp.float32)
    out = out + jnp.dot(x_ref[...], ws_ref[...], preferred_element_type=jnp.float32)
    out = out + b_ref[0][None, :]
    if act:
        out = jnp.maximum(out, 0.0)
    o_ref[...] = out


def _conv_skip_body(acc_ref, x_ref, skip_ref, wm_ref, ws_ref, b_ref, o_ref, *, act):
    agg = acc_ref[0] + acc_ref[1]
    out = jnp.dot(agg, wm_ref[...], preferred_element_type=jnp.float32)
    out = out + jnp.dot(x_ref[...], ws_ref[...], preferred_element_type=jnp.float32)
    out = out + b_ref[0][None, :]
    if act:
        out = jnp.maximum(out, 0.0)
    o_ref[...] = out + skip_ref[...]


def _make_conv(act, skip):
    body = _conv_skip_body if skip else _conv_body
    grid = NP // CONV_BLK
    mat_spec = pl.BlockSpec((CONV_BLK, D), lambda i: (i, 0))
    in_specs = [pl.BlockSpec((2, CONV_BLK, D), lambda i: (0, i, 0)), mat_spec]
    if skip:
        in_specs.append(mat_spec)
    in_specs += [
        pl.BlockSpec((D, D), lambda i: (0, 0)),
        pl.BlockSpec((D, D), lambda i: (0, 0)),
        pl.BlockSpec((8, D), lambda i: (0, 0)),
    ]
    return pl.pallas_call(
        functools.partial(body, act=act),
        grid=(grid,),
        in_specs=in_specs,
        out_specs=mat_spec,
        out_shape=jax.ShapeDtypeStruct((NP, D), jnp.float32),
    )


_conv_act = _make_conv(True, False)
_conv_act_skip = _make_conv(True, True)
_conv_noact = _make_conv(False, False)


def _comb(X, Y):
    out = []
    for i in range(3):
        for j in range(3):
            out.append(X[3 * i] * Y[j] + X[3 * i + 1] * Y[3 + j] + X[3 * i + 2] * Y[6 + j])
    for i in range(3):
        out.append(X[3 * i] * Y[9] + X[3 * i + 1] * Y[10] + X[3 * i + 2] * Y[11] + X[9 + i])
    return out


def _sel(dep, oprs):
    out = list(oprs[0])
    for j in range(1, MAX_RIGID):
        m = dep == float(j)
        for c in range(12):
            out[c] = jnp.where(m, oprs[j][c], out[c])
    return out


def _structure_body(bb_ref, sc_ref, pos_ref, g_ref, o_ref):
    def g(i):
        return g_ref[i]

    q1, q2, q3 = bb_ref[0], bb_ref[1], bb_ref[2]
    inv = lax.rsqrt(1.0 + q1 * q1 + q2 * q2 + q3 * q3)
    q0 = inv
    q1 = q1 * inv
    q2 = q2 * inv
    q3 = q3 * inv
    R = [
        q0 * q0 + q1 * q1 - q2 * q2 - q3 * q3, 2 * (q1 * q2 - q0 * q3), 2 * (q1 * q3 + q0 * q2),
        2 * (q1 * q2 + q0 * q3), q0 * q0 - q1 * q1 + q2 * q2 - q3 * q3, 2 * (q2 * q3 - q0 * q1),
        2 * (q1 * q3 - q0 * q2), 2 * (q2 * q3 + q0 * q1), q0 * q0 - q1 * q1 - q2 * q2 + q3 * q3,
    ]
    zeros = jnp.zeros_like(q0)
    ones = zeros + 1.0
    local = [R + [bb_ref[3] + pos_ref[0], bb_ref[4] + pos_ref[1], bb_ref[5] + pos_ref[2]]]
    for t in range(MAX_RIGID - 1):
        s_ = sc_ref[2 * t]
        c_ = sc_ref[2 * t + 1]
        ninv = lax.rsqrt(s_ * s_ + c_ * c_)
        sine = s_ * ninv
        cosine = c_ * ninv
        local.append([ones, zeros, zeros,
                      zeros, cosine, -sine,
                      zeros, sine, cosine,
                      zeros, zeros, zeros])
    opr = []
    for k in range(MAX_RIGID):
        X = [g(12 * k + c) for c in range(12)]
        opr.append(_comb(X, local[k]))
    for i_tor in range(1, MAX_RIGID):
        dep = g(138 + i_tor)
        prev = _sel(dep, opr)
        opr[i_tor] = _comb(prev, opr[i_tor])
    for a in range(NATOM):
        dep = g(146 + a)
        T = _sel(dep, opr)
        x = g(96 + 3 * a)
        y = g(96 + 3 * a + 1)
        z = g(96 + 3 * a + 2)
        for i in range(3):
            o_ref[3 * a + i] = T[3 * i] * x + T[3 * i + 1] * y + T[3 * i + 2] * z + T[9 + i]


_structure = pl.pallas_call(
    _structure_body,
    out_shape=jax.ShapeDtypeStruct((3 * NATOM, 8, NP // 8), jnp.float32),
)


def _pad_rows(a, rows):
    return jnp.concatenate(
        [a, jnp.zeros((rows - a.shape[0],) + a.shape[1:], a.dtype)], axis=0)


def _to_tiles(a):
    return a.T.reshape(a.shape[1], 8, NP // 8)


def _module(x, src_p, dst_p, zeros_hbm, p, skip=True):
    def conv(h, w, act, sk=None):
        agg = _seg_sum(h, src_p, dst_p, zeros_hbm)
        wm, ws, b = w
        b_pad = jnp.zeros((8, D), jnp.float32).at[0, : b.shape[0]].set(b)
        wm_p = jnp.zeros((D, D), jnp.float32).at[:, : wm.shape[1]].set(wm)
        ws_p = jnp.zeros((D, D), jnp.float32).at[:, : ws.shape[1]].set(ws)
        if sk is not None:
            return _conv_act_skip(agg, h, sk, wm_p, ws_p, b_pad)
        if act:
            return _conv_act(agg, h, wm_p, ws_p, b_pad)
        return _conv_noact(agg, h, wm_p, ws_p, b_pad)

    h = conv(x, p["l0"], True)
    for _ in range(2):
        h = conv(h, p["ls"], True, sk=h if skip else None)
    return conv(h, p["l1"], False)


def kernel(f_in0, f_in1, pos, params, rigid_transforms, rigid_groups,
           edge_index, residue_type, transforms_dep, groups_dep):
    src = edge_index[0].astype(jnp.int32)
    dst = edge_index[1].astype(jnp.int32)
    pad = E_PAD - E
    src_p = jnp.concatenate([src, jnp.zeros((pad,), jnp.int32)]).reshape(
        NW, CHUNKS_PER_W, EDGE_CHUNK)
    dst_p = jnp.concatenate([dst, jnp.full((pad,), N, jnp.int32)]).reshape(
        NW, CHUNKS_PER_W, EDGE_CHUNK)
    zeros_hbm = jnp.zeros((ROWS_PER_SUB, D), jnp.float32)

    x = jnp.concatenate([f_in0, f_in1.reshape(N, -1)], axis=1)
    x = _pad_rows(x, NP)

    f_out = _module(x, src_p, dst_p, zeros_hbm, params["fe"])
    bb_full = _module(f_out, src_p, dst_p, zeros_hbm, params["bb"])
    sc_full = _module(f_out, src_p, dst_p, zeros_hbm, params["sc"])
    bb = bb_full[:N, :6]
    sc = sc_full[:N, :14]

    tbl = jnp.concatenate([
        rigid_transforms.reshape(NRES, 96),
        rigid_groups.reshape(NRES, 42),
        transforms_dep.astype(jnp.float32),
        groups_dep.astype(jnp.float32),
        jnp.zeros((NRES, TBL_P - TBL), jnp.float32),
    ], axis=1)
    ridx = _pad_rows(residue_type.astype(jnp.int32), NP).reshape(NW, GCHUNKS, GCHUNK)
    g = _table_gather(tbl, ridx)[:, :TBL]

    rt = _structure(
        _to_tiles(bb_full[:, :6]),
        _to_tiles(sc_full[:, :14]),
        _to_tiles(_pad_rows(pos, NP)),
        _to_tiles(g),
    )
    R = rt.reshape(3 * NATOM, NP)[:, :N].T.reshape(N, NATOM, 3)
    return (bb, sc, R)


# speedup vs baseline: 6.1461x; 2.1921x over previous
import functools

import jax
import jax.numpy as jnp
from jax import lax
from jax.experimental import pallas as pl
from jax.experimental.pallas import tpu as pltpu
from jax.experimental.pallas import tpu_sc as plsc

N = 10000
E = 320000
D = 128
NRES = 21
MAX_RIGID = 8
NATOM = 14

NC = 2
NS = 16
NW = NC * NS

NP = 10240
ROWS_PER_SUB = NP // NS

DH = D // 2
EDGE_CHUNK = 128
CHUNKS_PER_T = 160
NBUF = 1
LA = 0
E_PAD = NS * CHUNKS_PER_T * EDGE_CHUNK
SH_ROWS = 10048
SH_PER_SUB = SH_ROWS // NS

GROWS_PER_W = NP // NW
GCHUNK = 80
GCHUNKS = GROWS_PER_W // GCHUNK
TBL = 96 + 42 + 8 + 14
TBL_P = 256

def _sc_mesh():
    return plsc.VectorSubcoreMesh(
        core_axis_name="c", subcore_axis_name="s", num_cores=NC, num_subcores=NS)


@functools.cache
def _build_seg_sum():
    @functools.partial(
        pl.kernel,
        out_type=jax.ShapeDtypeStruct((NP, NC, DH), jnp.float32),
        mesh=_sc_mesh(),
        scratch_types=[
            pltpu.VMEM((CHUNKS_PER_T, EDGE_CHUNK), jnp.int32),
            pltpu.VMEM((CHUNKS_PER_T, EDGE_CHUNK), jnp.int32),
            [pltpu.VMEM((EDGE_CHUNK, DH), jnp.float32)] * NBUF,
            pltpu.VMEM_SHARED((SH_ROWS, DH), jnp.float32),
            pltpu.VMEM_SHARED((SH_ROWS, DH), jnp.float32),
            [pltpu.SemaphoreType.DMA] * NBUF,
            [pltpu.SemaphoreType.DMA] * NBUF,
        ],
        compiler_params=pltpu.CompilerParams(use_tc_tiling_on_sc=False),
    )
    def seg_sum(x_hbm, src_hbm, dst_hbm, zeros_hbm, out_hbm,
                src_v, dst_v, rows_v, x_sh, acc_sh, sem_g, sem_s):
        cid = lax.axis_index("c")
        sid = lax.axis_index("s")

        pltpu.sync_copy(
            x_hbm.at[pl.ds(sid * SH_PER_SUB, SH_PER_SUB), pl.ds(cid * DH, DH)],
            x_sh.at[pl.ds(sid * SH_PER_SUB, SH_PER_SUB)])
        pltpu.sync_copy(zeros_hbm, acc_sh.at[pl.ds(sid * SH_PER_SUB, SH_PER_SUB)])
        pltpu.sync_copy(src_hbm.at[sid], src_v)
        pltpu.sync_copy(dst_hbm.at[sid], dst_v)
        plsc.subcore_barrier()

        n_sgroups = (CHUNKS_PER_T + LA + NBUF - 1) // NBUF

        @pl.loop(0, n_sgroups)
        def _steps(g):
            for b in range(NBUF):
                step = g * NBUF + b
                s_chunk = step - LA
                s_slot = (b - LA) % NBUF

                @pl.when(step < CHUNKS_PER_T)
                def _gather():
                    @pl.when(step >= NBUF)
                    def _drain_prev_scatter():
                        pltpu.make_async_copy(
                            rows_v[b], acc_sh.at[dst_v.at[step - NBUF]],
                            sem_s[b]).wait()

                    pltpu.async_copy(
                        x_sh.at[src_v.at[step]], rows_v[b], sem_g[b])

                @pl.when(jnp.logical_and(s_chunk >= 0, s_chunk < CHUNKS_PER_T))
                def _scatter():
                    pltpu.make_async_copy(
                        x_sh.at[src_v.at[s_chunk]], rows_v[s_slot],
                        sem_g[s_slot]).wait()
                    pltpu.async_copy(
                        rows_v[s_slot], acc_sh.at[dst_v.at[s_chunk]],
                        sem_s[s_slot], add=True)

        for b in range(NBUF):
            pltpu.make_async_copy(
                rows_v[b], acc_sh.at[dst_v.at[CHUNKS_PER_T - NBUF + b]],
                sem_s[b]).wait()
        plsc.subcore_barrier()
        pltpu.sync_copy(
            acc_sh.at[pl.ds(sid * SH_PER_SUB, SH_PER_SUB)],
            out_hbm.at[pl.ds(sid * SH_PER_SUB, SH_PER_SUB), cid],
        )

    return seg_sum


def _seg_sum(x, src_p, dst_p, zeros_hbm):
    out = _build_seg_sum()(x, src_p, dst_p, zeros_hbm)
    return out.reshape(NP, D)


@functools.cache
def _build_table_gather():
    @functools.partial(
        pl.kernel,
        out_type=jax.ShapeDtypeStruct((NP, TBL_P), jnp.float32),
        mesh=_sc_mesh(),
        scratch_types=[
            pltpu.VMEM((GCHUNKS, GCHUNK), jnp.int32),
            pltpu.VMEM((GCHUNK, TBL_P), jnp.float32),
            pltpu.SemaphoreType.DMA,
        ],
    )
    def table_gather(tbl_hbm, idx_hbm, out_hbm, idx_v, rows_v, sem):
        cid = lax.axis_index("c")
        sid = lax.axis_index("s")
        wid = cid * NS + sid
        pltpu.sync_copy(idx_hbm.at[wid], idx_v)

        @pl.loop(0, GCHUNKS)
        def _chunks(c):
            pltpu.async_copy(tbl_hbm.at[idx_v.at[c]], rows_v, sem).wait()
            base = wid * GROWS_PER_W + c * GCHUNK
            pltpu.sync_copy(rows_v, out_hbm.at[pl.ds(base, GCHUNK)])

    return table_gather


def _table_gather(tbl, ridx):
    return _build_table_gather()(tbl, ridx)


CONV_BLK = 2048


def _conv_body(acc_ref, x_ref, wm_ref, ws_ref, b_ref, o_ref, *, act):
    out = jnp.dot(acc_ref[...], wm_ref[...], preferred_element_type=jnp.float32)
    out = out + jnp.dot(x_ref[...], ws_ref[...], preferred_element_type=jnp.float32)
    out = out + b_ref[0][None, :]
    if act:
        out = jnp.maximum(out, 0.0)
    o_ref[...] = out


def _conv_skip_body(acc_ref, x_ref, skip_ref, wm_ref, ws_ref, b_ref, o_ref, *, act):
    out = jnp.dot(acc_ref[...], wm_ref[...], preferred_element_type=jnp.float32)
    out = out + jnp.dot(x_ref[...], ws_ref[...], preferred_element_type=jnp.float32)
    out = out + b_ref[0][None, :]
    if act:
        out = jnp.maximum(out, 0.0)
    o_ref[...] = out + skip_ref[...]


def _make_conv(act, skip):
    body = _conv_skip_body if skip else _conv_body
    grid = NP // CONV_BLK
    mat_spec = pl.BlockSpec((CONV_BLK, D), lambda i: (i, 0))
    in_specs = [mat_spec, mat_spec]
    if skip:
        in_specs.append(mat_spec)
    in_specs += [
        pl.BlockSpec((D, D), lambda i: (0, 0)),
        pl.BlockSpec((D, D), lambda i: (0, 0)),
        pl.BlockSpec((8, D), lambda i: (0, 0)),
    ]
    return pl.pallas_call(
        functools.partial(body, act=act),
        grid=(grid,),
        in_specs=in_specs,
        out_specs=mat_spec,
        out_shape=jax.ShapeDtypeStruct((NP, D), jnp.float32),
    )


_conv_act = _make_conv(True, False)
_conv_act_skip = _make_conv(True, True)
_conv_noact = _make_conv(False, False)


def _comb(X, Y):
    out = []
    for i in range(3):
        for j in range(3):
            out.append(X[3 * i] * Y[j] + X[3 * i + 1] * Y[3 + j] + X[3 * i + 2] * Y[6 + j])
    for i in range(3):
        out.append(X[3 * i] * Y[9] + X[3 * i + 1] * Y[10] + X[3 * i + 2] * Y[11] + X[9 + i])
    return out


def _sel(dep, oprs):
    out = list(oprs[0])
    for j in range(1, MAX_RIGID):
        m = dep == float(j)
        for c in range(12):
            out[c] = jnp.where(m, oprs[j][c], out[c])
    return out


def _structure_body(bb_ref, sc_ref, pos_ref, g_ref, o_ref):
    def g(i):
        return g_ref[i]

    q1, q2, q3 = bb_ref[0], bb_ref[1], bb_ref[2]
    inv = lax.rsqrt(1.0 + q1 * q1 + q2 * q2 + q3 * q3)
    q0 = inv
    q1 = q1 * inv
    q2 = q2 * inv
    q3 = q3 * inv
    R = [
        q0 * q0 + q1 * q1 - q2 * q2 - q3 * q3, 2 * (q1 * q2 - q0 * q3), 2 * (q1 * q3 + q0 * q2),
        2 * (q1 * q2 + q0 * q3), q0 * q0 - q1 * q1 + q2 * q2 - q3 * q3, 2 * (q2 * q3 - q0 * q1),
        2 * (q1 * q3 - q0 * q2), 2 * (q2 * q3 + q0 * q1), q0 * q0 - q1 * q1 - q2 * q2 + q3 * q3,
    ]
    zeros = jnp.zeros_like(q0)
    ones = zeros + 1.0
    local = [R + [bb_ref[3] + pos_ref[0], bb_ref[4] + pos_ref[1], bb_ref[5] + pos_ref[2]]]
    for t in range(MAX_RIGID - 1):
        s_ = sc_ref[2 * t]
        c_ = sc_ref[2 * t + 1]
        ninv = lax.rsqrt(s_ * s_ + c_ * c_)
        sine = s_ * ninv
        cosine = c_ * ninv
        local.append([ones, zeros, zeros,
                      zeros, cosine, -sine,
                      zeros, sine, cosine,
                      zeros, zeros, zeros])
    opr = []
    for k in range(MAX_RIGID):
        X = [g(12 * k + c) for c in range(12)]
        opr.append(_comb(X, local[k]))
    for i_tor in range(1, MAX_RIGID):
        dep = g(138 + i_tor)
        prev = _sel(dep, opr)
        opr[i_tor] = _comb(prev, opr[i_tor])
    for a in range(NATOM):
        dep = g(146 + a)
        T = _sel(dep, opr)
        x = g(96 + 3 * a)
        y = g(96 + 3 * a + 1)
        z = g(96 + 3 * a + 2)
        for i in range(3):
            o_ref[3 * a + i] = T[3 * i] * x + T[3 * i + 1] * y + T[3 * i + 2] * z + T[9 + i]


_structure = pl.pallas_call(
    _structure_body,
    out_shape=jax.ShapeDtypeStruct((3 * NATOM, 8, NP // 8), jnp.float32),
)


def _pad_rows(a, rows):
    return jnp.concatenate(
        [a, jnp.zeros((rows - a.shape[0],) + a.shape[1:], a.dtype)], axis=0)


def _to_tiles(a):
    return a.T.reshape(a.shape[1], 8, NP // 8)


def _module(x, src_p, dst_p, zeros_hbm, p, skip=True):
    def conv(h, w, act, sk=None):
        agg = _seg_sum(h, src_p, dst_p, zeros_hbm)
        wm, ws, b = w
        b_pad = jnp.zeros((8, D), jnp.float32).at[0, : b.shape[0]].set(b)
        wm_p = jnp.zeros((D, D), jnp.float32).at[:, : wm.shape[1]].set(wm)
        ws_p = jnp.zeros((D, D), jnp.float32).at[:, : ws.shape[1]].set(ws)
        if sk is not None:
            return _conv_act_skip(agg, h, sk, wm_p, ws_p, b_pad)
        if act:
            return _conv_act(agg, h, wm_p, ws_p, b_pad)
        return _conv_noact(agg, h, wm_p, ws_p, b_pad)

    h = conv(x, p["l0"], True)
    for _ in range(2):
        h = conv(h, p["ls"], True, sk=h if skip else None)
    return conv(h, p["l1"], False)


def kernel(f_in0, f_in1, pos, params, rigid_transforms, rigid_groups,
           edge_index, residue_type, transforms_dep, groups_dep):
    src = edge_index[0].astype(jnp.int32)
    dst = edge_index[1].astype(jnp.int32)
    pad = E_PAD - E
    src_p = jnp.concatenate([src, jnp.zeros((pad,), jnp.int32)]).reshape(
        NS, CHUNKS_PER_T, EDGE_CHUNK)
    dst_p = jnp.concatenate([dst, jnp.full((pad,), N, jnp.int32)]).reshape(
        NS, CHUNKS_PER_T, EDGE_CHUNK)
    zeros_hbm = jnp.zeros((SH_PER_SUB, DH), jnp.float32)

    x = jnp.concatenate([f_in0, f_in1.reshape(N, -1)], axis=1)
    x = _pad_rows(x, NP)

    f_out = _module(x, src_p, dst_p, zeros_hbm, params["fe"])
    bb_full = _module(f_out, src_p, dst_p, zeros_hbm, params["bb"])
    sc_full = _module(f_out, src_p, dst_p, zeros_hbm, params["sc"])
    bb = bb_full[:N, :6]
    sc = sc_full[:N, :14]

    tbl = jnp.concatenate([
        rigid_transforms.reshape(NRES, 96),
        rigid_groups.reshape(NRES, 42),
        transforms_dep.astype(jnp.float32),
        groups_dep.astype(jnp.float32),
        jnp.zeros((NRES, TBL_P - TBL), jnp.float32),
    ], axis=1)
    ridx = _pad_rows(residue_type.astype(jnp.int32), NP).reshape(NW, GCHUNKS, GCHUNK)
    g = _table_gather(tbl, ridx)[:, :TBL]

    rt = _structure(
        _to_tiles(bb_full[:, :6]),
        _to_tiles(sc_full[:, :14]),
        _to_tiles(_pad_rows(pos, NP)),
        _to_tiles(g),
    )
    R = rt.reshape(3 * NATOM, NP)[:, :N].T.reshape(N, NATOM, 3)
    return (bb, sc, R)


# speedup vs baseline: 9.5007x; 1.5458x over previous
import functools

import jax
import jax.numpy as jnp
from jax import lax
from jax.experimental import pallas as pl
from jax.experimental.pallas import tpu as pltpu
from jax.experimental.pallas import tpu_sc as plsc

N = 10000
E = 320000
D = 128
NRES = 21
MAX_RIGID = 8
NATOM = 14

NC = 2
NS = 16
NW = NC * NS

NP = 10240
ROWS_PER_SUB = NP // NS

DH = D // 2
EDGE_CHUNK = 128
CHUNKS_PER_T = 160
NBUF = 3
LA = 1
HC = CHUNKS_PER_T // 2
E_PAD = NS * CHUNKS_PER_T * EDGE_CHUNK
SH_ROWS = 10048
SH_PER_SUB = SH_ROWS // NS

GROWS_PER_W = NP // NW
GCHUNK = 80
GCHUNKS = GROWS_PER_W // GCHUNK
TBL = 96 + 42 + 8 + 14
TBL_P = 256

def _sc_mesh():
    return plsc.VectorSubcoreMesh(
        core_axis_name="c", subcore_axis_name="s", num_cores=NC, num_subcores=NS)


@functools.cache
def _build_seg_sum():
    @functools.partial(
        pl.kernel,
        out_type=jax.ShapeDtypeStruct((NP, NC, DH), jnp.float32),
        mesh=_sc_mesh(),
        scratch_types=[
            pltpu.VMEM((HC, EDGE_CHUNK), jnp.int32),
            pltpu.VMEM((HC, EDGE_CHUNK), jnp.int32),
            [pltpu.VMEM((EDGE_CHUNK, DH), jnp.float32)] * NBUF,
            pltpu.VMEM_SHARED((SH_ROWS, DH), jnp.float32),
            pltpu.VMEM_SHARED((SH_ROWS, DH), jnp.float32),
            [pltpu.SemaphoreType.DMA] * NBUF,
            [pltpu.SemaphoreType.DMA] * NBUF,
        ],
        compiler_params=pltpu.CompilerParams(use_tc_tiling_on_sc=False),
    )
    def seg_sum(x_hbm, src_hbm, dst_hbm, zeros_hbm, out_hbm,
                src_v, dst_v, rows_v, x_sh, acc_sh, sem_g, sem_s):
        cid = lax.axis_index("c")
        sid = lax.axis_index("s")

        pltpu.sync_copy(
            x_hbm.at[pl.ds(sid * SH_PER_SUB, SH_PER_SUB), pl.ds(cid * DH, DH)],
            x_sh.at[pl.ds(sid * SH_PER_SUB, SH_PER_SUB)])
        pltpu.sync_copy(zeros_hbm, acc_sh.at[pl.ds(sid * SH_PER_SUB, SH_PER_SUB)])
        plsc.subcore_barrier()

        n_sgroups = (HC + LA + NBUF - 1) // NBUF
        for p in range(CHUNKS_PER_T // HC):
            pltpu.sync_copy(src_hbm.at[sid, pl.ds(p * HC, HC)], src_v)
            pltpu.sync_copy(dst_hbm.at[sid, pl.ds(p * HC, HC)], dst_v)

            @pl.loop(0, n_sgroups)
            def _steps(g):
                for b in range(NBUF):
                    step = g * NBUF + b
                    s_chunk = step - LA
                    s_slot = (b - LA) % NBUF

                    @pl.when(step < HC)
                    def _gather():
                        @pl.when(step >= NBUF)
                        def _drain_prev_scatter():
                            pltpu.make_async_copy(
                                rows_v[b], acc_sh.at[dst_v.at[step - NBUF]],
                                sem_s[b]).wait()

                        pltpu.async_copy(
                            x_sh.at[src_v.at[step]], rows_v[b], sem_g[b])

                    @pl.when(jnp.logical_and(s_chunk >= 0, s_chunk < HC))
                    def _scatter():
                        pltpu.make_async_copy(
                            x_sh.at[src_v.at[s_chunk]], rows_v[s_slot],
                            sem_g[s_slot]).wait()
                        pltpu.async_copy(
                            rows_v[s_slot], acc_sh.at[dst_v.at[s_chunk]],
                            sem_s[s_slot], add=True)

            for b in range(NBUF):
                c_last = ((HC - 1 - b) // NBUF) * NBUF + b
                pltpu.make_async_copy(
                    rows_v[b], acc_sh.at[dst_v.at[c_last]], sem_s[b]).wait()
        plsc.subcore_barrier()
        pltpu.sync_copy(
            acc_sh.at[pl.ds(sid * SH_PER_SUB, SH_PER_SUB)],
            out_hbm.at[pl.ds(sid * SH_PER_SUB, SH_PER_SUB), cid],
        )

    return seg_sum


def _seg_sum(x, src_p, dst_p, zeros_hbm):
    out = _build_seg_sum()(x, src_p, dst_p, zeros_hbm)
    return out.reshape(NP, D)


@functools.cache
def _build_table_gather():
    @functools.partial(
        pl.kernel,
        out_type=jax.ShapeDtypeStruct((NP, TBL_P), jnp.float32),
        mesh=_sc_mesh(),
        scratch_types=[
            pltpu.VMEM((GCHUNKS, GCHUNK), jnp.int32),
            pltpu.VMEM((GCHUNK, TBL_P), jnp.float32),
            pltpu.SemaphoreType.DMA,
        ],
    )
    def table_gather(tbl_hbm, idx_hbm, out_hbm, idx_v, rows_v, sem):
        cid = lax.axis_index("c")
        sid = lax.axis_index("s")
        wid = cid * NS + sid
        pltpu.sync_copy(idx_hbm.at[wid], idx_v)

        @pl.loop(0, GCHUNKS)
        def _chunks(c):
            pltpu.async_copy(tbl_hbm.at[idx_v.at[c]], rows_v, sem).wait()
            base = wid * GROWS_PER_W + c * GCHUNK
            pltpu.sync_copy(rows_v, out_hbm.at[pl.ds(base, GCHUNK)])

    return table_gather


def _table_gather(tbl, ridx):
    return _build_table_gather()(tbl, ridx)


CONV_BLK = 2048


def _conv_body(acc_ref, x_ref, wm_ref, ws_ref, b_ref, o_ref, *, act):
    out = jnp.dot(acc_ref[...], wm_ref[...], preferred_element_type=jnp.float32)
    out = out + jnp.dot(x_ref[...], ws_ref[...], preferred_element_type=jnp.float32)
    out = out + b_ref[0][None, :]
    if act:
        out = jnp.maximum(out, 0.0)
    o_ref[...] = out


def _conv_skip_body(acc_ref, x_ref, skip_ref, wm_ref, ws_ref, b_ref, o_ref, *, act):
    out = jnp.dot(acc_ref[...], wm_ref[...], preferred_element_type=jnp.float32)
    out = out + jnp.dot(x_ref[...], ws_ref[...], preferred_element_type=jnp.float32)
    out = out + b_ref[0][None, :]
    if act:
        out = jnp.maximum(out, 0.0)
    o_ref[...] = out + skip_ref[...]


def _make_conv(act, skip):
    body = _conv_skip_body if skip else _conv_body
    grid = NP // CONV_BLK
    mat_spec = pl.BlockSpec((CONV_BLK, D), lambda i: (i, 0))
    in_specs = [mat_spec, mat_spec]
    if skip:
        in_specs.append(mat_spec)
    in_specs += [
        pl.BlockSpec((D, D), lambda i: (0, 0)),
        pl.BlockSpec((D, D), lambda i: (0, 0)),
        pl.BlockSpec((8, D), lambda i: (0, 0)),
    ]
    return pl.pallas_call(
        functools.partial(body, act=act),
        grid=(grid,),
        in_specs=in_specs,
        out_specs=mat_spec,
        out_shape=jax.ShapeDtypeStruct((NP, D), jnp.float32),
    )


_conv_act = _make_conv(True, False)
_conv_act_skip = _make_conv(True, True)
_conv_noact = _make_conv(False, False)


def _comb(X, Y):
    out = []
    for i in range(3):
        for j in range(3):
            out.append(X[3 * i] * Y[j] + X[3 * i + 1] * Y[3 + j] + X[3 * i + 2] * Y[6 + j])
    for i in range(3):
        out.append(X[3 * i] * Y[9] + X[3 * i + 1] * Y[10] + X[3 * i + 2] * Y[11] + X[9 + i])
    return out


def _sel(dep, oprs):
    out = list(oprs[0])
    for j in range(1, MAX_RIGID):
        m = dep == float(j)
        for c in range(12):
            out[c] = jnp.where(m, oprs[j][c], out[c])
    return out


def _structure_body(bb_ref, sc_ref, pos_ref, g_ref, o_ref):
    def g(i):
        return g_ref[i]

    q1, q2, q3 = bb_ref[0], bb_ref[1], bb_ref[2]
    inv = lax.rsqrt(1.0 + q1 * q1 + q2 * q2 + q3 * q3)
    q0 = inv
    q1 = q1 * inv
    q2 = q2 * inv
    q3 = q3 * inv
    R = [
        q0 * q0 + q1 * q1 - q2 * q2 - q3 * q3, 2 * (q1 * q2 - q0 * q3), 2 * (q1 * q3 + q0 * q2),
        2 * (q1 * q2 + q0 * q3), q0 * q0 - q1 * q1 + q2 * q2 - q3 * q3, 2 * (q2 * q3 - q0 * q1),
        2 * (q1 * q3 - q0 * q2), 2 * (q2 * q3 + q0 * q1), q0 * q0 - q1 * q1 - q2 * q2 + q3 * q3,
    ]
    zeros = jnp.zeros_like(q0)
    ones = zeros + 1.0
    local = [R + [bb_ref[3] + pos_ref[0], bb_ref[4] + pos_ref[1], bb_ref[5] + pos_ref[2]]]
    for t in range(MAX_RIGID - 1):
        s_ = sc_ref[2 * t]
        c_ = sc_ref[2 * t + 1]
        ninv = lax.rsqrt(s_ * s_ + c_ * c_)
        sine = s_ * ninv
        cosine = c_ * ninv
        local.append([ones, zeros, zeros,
                      zeros, cosine, -sine,
                      zeros, sine, cosine,
                      zeros, zeros, zeros])
    opr = []
    for k in range(MAX_RIGID):
        X = [g(12 * k + c) for c in range(12)]
        opr.append(_comb(X, local[k]))
    for i_tor in range(1, MAX_RIGID):
        dep = g(138 + i_tor)
        prev = _sel(dep, opr)
        opr[i_tor] = _comb(prev, opr[i_tor])
    for a in range(NATOM):
        dep = g(146 + a)
        T = _sel(dep, opr)
        x = g(96 + 3 * a)
        y = g(96 + 3 * a + 1)
        z = g(96 + 3 * a + 2)
        for i in range(3):
            o_ref[3 * a + i] = T[3 * i] * x + T[3 * i + 1] * y + T[3 * i + 2] * z + T[9 + i]


_structure = pl.pallas_call(
    _structure_body,
    out_shape=jax.ShapeDtypeStruct((3 * NATOM, 8, NP // 8), jnp.float32),
)


def _pad_rows(a, rows):
    return jnp.concatenate(
        [a, jnp.zeros((rows - a.shape[0],) + a.shape[1:], a.dtype)], axis=0)


def _to_tiles(a):
    return a.T.reshape(a.shape[1], 8, NP // 8)


def _module(x, src_p, dst_p, zeros_hbm, p, skip=True):
    def conv(h, w, act, sk=None):
        agg = _seg_sum(h, src_p, dst_p, zeros_hbm)
        wm, ws, b = w
        b_pad = jnp.zeros((8, D), jnp.float32).at[0, : b.shape[0]].set(b)
        wm_p = jnp.zeros((D, D), jnp.float32).at[:, : wm.shape[1]].set(wm)
        ws_p = jnp.zeros((D, D), jnp.float32).at[:, : ws.shape[1]].set(ws)
        if sk is not None:
            return _conv_act_skip(agg, h, sk, wm_p, ws_p, b_pad)
        if act:
            return _conv_act(agg, h, wm_p, ws_p, b_pad)
        return _conv_noact(agg, h, wm_p, ws_p, b_pad)

    h = conv(x, p["l0"], True)
    for _ in range(2):
        h = conv(h, p["ls"], True, sk=h if skip else None)
    return conv(h, p["l1"], False)


def kernel(f_in0, f_in1, pos, params, rigid_transforms, rigid_groups,
           edge_index, residue_type, transforms_dep, groups_dep):
    src = edge_index[0].astype(jnp.int32)
    dst = edge_index[1].astype(jnp.int32)
    pad = E_PAD - E
    src_p = jnp.concatenate([src, jnp.zeros((pad,), jnp.int32)]).reshape(
        NS, CHUNKS_PER_T, EDGE_CHUNK)
    dst_p = jnp.concatenate([dst, jnp.full((pad,), N, jnp.int32)]).reshape(
        NS, CHUNKS_PER_T, EDGE_CHUNK)
    zeros_hbm = jnp.zeros((SH_PER_SUB, DH), jnp.float32)

    x = jnp.concatenate([f_in0, f_in1.reshape(N, -1)], axis=1)
    x = _pad_rows(x, NP)

    f_out = _module(x, src_p, dst_p, zeros_hbm, params["fe"])
    bb_full = _module(f_out, src_p, dst_p, zeros_hbm, params["bb"])
    sc_full = _module(f_out, src_p, dst_p, zeros_hbm, params["sc"])
    bb = bb_full[:N, :6]
    sc = sc_full[:N, :14]

    tbl = jnp.concatenate([
        rigid_transforms.reshape(NRES, 96),
        rigid_groups.reshape(NRES, 42),
        transforms_dep.astype(jnp.float32),
        groups_dep.astype(jnp.float32),
        jnp.zeros((NRES, TBL_P - TBL), jnp.float32),
    ], axis=1)
    ridx = _pad_rows(residue_type.astype(jnp.int32), NP).reshape(NW, GCHUNKS, GCHUNK)
    g = _table_gather(tbl, ridx)[:, :TBL]

    rt = _structure(
        _to_tiles(bb_full[:, :6]),
        _to_tiles(sc_full[:, :14]),
        _to_tiles(_pad_rows(pos, NP)),
        _to_tiles(g),
    )
    R = rt.reshape(3 * NATOM, NP)[:, :N].T.reshape(N, NATOM, 3)
    return (bb, sc, R)
